# Initial kernel scaffold; baseline (speedup 1.0000x reference)
import functools

import jax
import jax.numpy as jnp
from jax import lax
from jax.experimental import pallas as pl
from jax.experimental.pallas import tpu as pltpu
from jax.experimental.pallas import tpu_sc as plsc

EMBED_DIM = 64
NUM_WORKERS = 32
CHUNK = 128


@functools.partial(jax.jit, static_argnames=("n_chunks",))
def _gather(weight, idx, n_chunks):
    b_total = idx.size
    b_per_w = b_total // NUM_WORKERS
    mesh = plsc.VectorSubcoreMesh(core_axis_name="c", subcore_axis_name="s")

    @functools.partial(
        pl.kernel,
        mesh=mesh,
        out_type=jax.ShapeDtypeStruct((b_total, EMBED_DIM), jnp.float32),
        scratch_types=[
            pltpu.VMEM((n_chunks, CHUNK), jnp.int32),
            pltpu.VMEM((CHUNK, EMBED_DIM), jnp.float32),
            pltpu.SemaphoreType.DMA,
        ],
    )
    def body(weight_hbm, idx_hbm, out_hbm, idx_v, rows_v, sem):
        wid = lax.axis_index("s") * 2 + lax.axis_index("c")
        base = wid * b_per_w
        pltpu.sync_copy(idx_hbm.at[wid], idx_v)

        def step(j, carry):
            pltpu.async_copy(weight_hbm.at[idx_v.at[j]], rows_v, sem).wait()
            pltpu.sync_copy(
                rows_v, out_hbm.at[pl.ds(base + j * CHUNK, CHUNK)])
            return carry

        lax.fori_loop(0, n_chunks, step, 0)

    return body(weight, idx)


def kernel(x, weight):
    b_total = x.size
    b_per_w = b_total // NUM_WORKERS
    n_chunks = b_per_w // CHUNK
    idx = x.reshape(NUM_WORKERS, n_chunks, CHUNK).astype(jnp.int32)
    out = _gather(weight, idx, n_chunks)
    return out.reshape(x.shape + (EMBED_DIM,))


# speedup vs baseline: 4.0804x; 4.0804x over previous
import functools

import jax
import jax.numpy as jnp
from jax import lax
from jax.experimental import pallas as pl
from jax.experimental.pallas import tpu as pltpu
from jax.experimental.pallas import tpu_sc as plsc

EMBED_DIM = 64
NUM_WORKERS = 32
CHUNK = 128


@functools.partial(jax.jit, static_argnames=("n_chunks",))
def _gather(weight, idx, n_chunks):
    b_total = idx.size
    b_per_w = b_total // NUM_WORKERS
    mesh = plsc.VectorSubcoreMesh(core_axis_name="c", subcore_axis_name="s")

    @functools.partial(
        pl.kernel,
        mesh=mesh,
        out_type=jax.ShapeDtypeStruct((b_total, EMBED_DIM), jnp.float32),
        scratch_types=[
            pltpu.VMEM((n_chunks, CHUNK), jnp.int32),
            pltpu.VMEM((CHUNK, EMBED_DIM), jnp.float32),
            pltpu.SemaphoreType.DMA,
        ],
        compiler_params=pltpu.CompilerParams(use_tc_tiling_on_sc=False),
    )
    def body(weight_hbm, idx_hbm, out_hbm, idx_v, rows_v, sem):
        wid = lax.axis_index("s") * 2 + lax.axis_index("c")
        base = wid * b_per_w
        pltpu.sync_copy(idx_hbm.at[wid], idx_v)

        def step(j, carry):
            pltpu.async_copy(weight_hbm.at[idx_v.at[j]], rows_v, sem).wait()
            pltpu.sync_copy(
                rows_v, out_hbm.at[pl.ds(base + j * CHUNK, CHUNK)])
            return carry

        lax.fori_loop(0, n_chunks, step, 0)

    return body(weight, idx)


def kernel(x, weight):
    b_total = x.size
    b_per_w = b_total // NUM_WORKERS
    n_chunks = b_per_w // CHUNK
    idx = x.reshape(NUM_WORKERS, n_chunks, CHUNK).astype(jnp.int32)
    out = _gather(weight, idx, n_chunks)
    return out.reshape(x.shape + (EMBED_DIM,))


# speedup vs baseline: 4.6663x; 1.1436x over previous
import functools

import jax
import jax.numpy as jnp
from jax import lax
from jax.experimental import pallas as pl
from jax.experimental.pallas import tpu as pltpu
from jax.experimental.pallas import tpu_sc as plsc

EMBED_DIM = 64
NUM_WORKERS = 32
CHUNK = 128
NBUF = 5


@functools.partial(jax.jit, static_argnames=("n_chunks",))
def _gather(weight, idx, n_chunks):
    b_total = idx.size
    b_per_w = b_total // NUM_WORKERS
    n_groups = n_chunks // NBUF
    mesh = plsc.VectorSubcoreMesh(core_axis_name="c", subcore_axis_name="s")

    @functools.partial(
        pl.kernel,
        mesh=mesh,
        out_type=jax.ShapeDtypeStruct((b_total, EMBED_DIM), jnp.float32),
        scratch_types=[
            pltpu.VMEM((n_chunks, CHUNK), jnp.int32),
            [pltpu.VMEM((CHUNK, EMBED_DIM), jnp.float32)
             for _ in range(NBUF)],
            [pltpu.SemaphoreType.DMA for _ in range(NBUF)],
            [pltpu.SemaphoreType.DMA for _ in range(NBUF)],
        ],
        compiler_params=pltpu.CompilerParams(use_tc_tiling_on_sc=False),
    )
    def body(weight_hbm, idx_hbm, out_hbm, idx_v, rows, gsems, wsems):
        wid = lax.axis_index("s") * 2 + lax.axis_index("c")
        base = wid * b_per_w
        pltpu.sync_copy(idx_hbm.at[wid], idx_v)

        def gather_copy(j, b):
            return pltpu.make_async_copy(
                weight_hbm.at[idx_v.at[j]], rows[b], gsems[b])

        def write_copy(j, b):
            return pltpu.make_async_copy(
                rows[b], out_hbm.at[pl.ds(base + j * CHUNK, CHUNK)],
                wsems[b])

        for b in range(NBUF):
            gather_copy(b, b).start()

        def group(g, carry):
            j0 = g * NBUF
            for b in range(NBUF):
                gather_copy(j0 + b, b).wait()
                write_copy(j0 + b, b).start()
            @pl.when(g + 1 < n_groups)
            def _():
                for b in range(NBUF):
                    write_copy(j0 + b, b).wait()
                    gather_copy(j0 + NBUF + b, b).start()
            return carry

        lax.fori_loop(0, n_groups, group, 0)

        for b in range(NBUF):
            write_copy((n_groups - 1) * NBUF + b, b).wait()

    return body(weight, idx)


def kernel(x, weight):
    b_total = x.size
    b_per_w = b_total // NUM_WORKERS
    n_chunks = b_per_w // CHUNK
    idx = x.reshape(NUM_WORKERS, n_chunks, CHUNK).astype(jnp.int32)
    out = _gather(weight, idx, n_chunks)
    return out.reshape(x.shape + (EMBED_DIM,))
